# baseline (speedup 1.0000x reference)
import functools

import jax
import jax.numpy as jnp
from jax import lax
from jax.experimental import pallas as pl
from jax.experimental.pallas import tpu as pltpu
from jax.experimental.pallas import tpu_sc as plsc

N = 10000
E = 160000
H = 128
NRBF = 16

NC = 2
NS = 16
EPT = E // NS
NCHUNK = 8
C = 1280
RPT = C // NS
SCRAP = C
K = 16
GRP = 16
SEG = 2000
W = 512


def _silu(x):
    return x * (1.0 / (1.0 + jnp.exp(-x)))


def _ctx_body(inv_ref, lnw_ref, lnb_ref, w1t_ref, b1_ref, w2t_ref, b2_ref,
              s_ref, out_ref):
    x = s_ref[...]
    mu = jnp.mean(x, axis=1, keepdims=True)
    xc = x - mu
    var = jnp.mean(xc * xc, axis=1, keepdims=True)
    y = xc * lax.rsqrt(var + 1e-5) * lnw_ref[...] + lnb_ref[...]
    h1 = jnp.dot(y, w1t_ref[...], preferred_element_type=jnp.float32)
    h1 = h1 + b1_ref[...]
    h2 = jnp.dot(_silu(h1), w2t_ref[...], preferred_element_type=jnp.float32)
    out_ref[...] = (h2 + b2_ref[...]) * inv_ref[0, 0]


def _ctx_tc(s, inv, ln_w, ln_b, w1, b1, w2, b2):
    BA = 400
    grid = N // BA
    return pl.pallas_call(
        _ctx_body,
        grid=(grid,),
        in_specs=[
            pl.BlockSpec(memory_space=pltpu.SMEM),
            pl.BlockSpec((1, H), lambda i: (0, 0)),
            pl.BlockSpec((1, H), lambda i: (0, 0)),
            pl.BlockSpec((H, H), lambda i: (0, 0)),
            pl.BlockSpec((1, H), lambda i: (0, 0)),
            pl.BlockSpec((H, 3 * H), lambda i: (0, 0)),
            pl.BlockSpec((1, 3 * H), lambda i: (0, 0)),
            pl.BlockSpec((BA, H), lambda i: (i, 0)),
        ],
        out_specs=pl.BlockSpec((BA, 3 * H), lambda i: (i, 0)),
        out_shape=jax.ShapeDtypeStruct((N, 3 * H), jnp.float32),
    )(inv.reshape(1, 1), ln_w.reshape(1, H), ln_b.reshape(1, H),
      w1.T, b1.reshape(1, H), w2.T, b2.reshape(1, 3 * H), s)


def _w640_body(f1t_ref, fb1_ref, f2t_ref, fb2_ref, rbf_ref, fc_ref, dir_ref,
               out_ref):
    h1 = jnp.dot(rbf_ref[...], f1t_ref[...],
                 preferred_element_type=jnp.float32) + fb1_ref[...]
    g = jnp.dot(_silu(h1), f2t_ref[...],
                preferred_element_type=jnp.float32) + fb2_ref[...]
    fc = fc_ref[...]
    ga = g[:, :H] * fc
    gb = g[:, H:2 * H] * fc
    gc = g[:, 2 * H:] * fc
    be = ga.shape[0]
    d0 = jnp.broadcast_to(dir_ref[:, 0:1], (be, GRP))
    d1 = jnp.broadcast_to(dir_ref[:, 1:2], (be, GRP))
    d2 = jnp.broadcast_to(dir_ref[:, 2:3], (be, GRP))
    pad = jnp.zeros((be, W - 3 * H - 3 * GRP), jnp.float32)
    out_ref[...] = jnp.concatenate([ga, gb, gc, d0, d1, d2, pad], axis=1)


def _w640_tc(rbf, f_cut, dir_ij, f1, fb1, f2, fb2):
    BE = 1000
    grid = E // BE
    return pl.pallas_call(
        _w640_body,
        grid=(grid,),
        in_specs=[
            pl.BlockSpec((NRBF, H), lambda i: (0, 0)),
            pl.BlockSpec((1, H), lambda i: (0, 0)),
            pl.BlockSpec((H, 3 * H), lambda i: (0, 0)),
            pl.BlockSpec((1, 3 * H), lambda i: (0, 0)),
            pl.BlockSpec((BE, NRBF), lambda i: (i, 0)),
            pl.BlockSpec((BE, 1), lambda i: (i, 0)),
            pl.BlockSpec((BE, 3), lambda i: (i, 0)),
        ],
        out_specs=pl.BlockSpec((BE, W), lambda i: (i, 0)),
        out_shape=jax.ShapeDtypeStruct((E, W), jnp.float32),
    )(f1.T, fb1.reshape(1, H), f2.T, fb2.reshape(1, 3 * H),
      rbf, f_cut.reshape(E, 1), dir_ij)


def _sc_body(w640, cv, idxi, idxj, su, vu,
             idxi_b, idxj_b, sel_b, scanbuf, jidx, eidx, ridx,
             vi0, vi1, vi2, wbuf, cvbuf, osb, ovb,
             acc_s, acc_v, gs0, gs1, ss0, ss1):
    c = lax.axis_index("c")
    s = lax.axis_index("s")
    base_e = s * EPT
    z16 = jnp.zeros((GRP,), jnp.float32)
    iota16 = lax.iota(jnp.int32, GRP)

    def _chunk_body(pc, chunk_carry):
        chunk = c * (NCHUNK // NC) + pc
        lo = chunk * C
        r0 = s * RPT

        def _zero_row(i, carry):
            for h in range(H // GRP):
                osb[0, i, pl.ds(h * GRP, GRP)] = z16
            for cc in range(3):
                for h in range(H // GRP):
                    ovb[0, cc, i, pl.ds(h * GRP, GRP)] = z16
            return carry

        lax.fori_loop(0, K, _zero_row, 0)

        def _copies(total):
            out, off = [], 0
            while off < total:
                n = min(K, total - off)
                out.append((off, n))
                off += n
            return out

        for off, n in _copies(RPT):
            pltpu.sync_copy(osb.at[0].at[pl.ds(0, n)],
                            acc_s.at[pl.ds(r0 + off, n)])
        for off, n in _copies(3 * RPT):
            pltpu.sync_copy(ovb.at[0].at[0].at[pl.ds(0, n)],
                            acc_v.at[pl.ds(3 * r0 + off, n)])

        @pl.when(s == 0)
        def _zero_scrap():
            pltpu.sync_copy(osb.at[0].at[pl.ds(0, 1)],
                            acc_s.at[pl.ds(SCRAP, 1)])
            pltpu.sync_copy(ovb.at[0].at[0].at[pl.ds(0, 3)],
                            acc_v.at[pl.ds(3 * SCRAP, 3)])

        plsc.subcore_barrier()

        splat15 = jnp.full((GRP,), 15, jnp.int32)

        def _scat_wait(slot, ss):
            pltpu.make_async_copy(osb.at[slot],
                                  acc_s.at[pl.ds(0, K)], ss).wait()
            for cc in range(3):
                pltpu.make_async_copy(ovb.at[slot].at[cc],
                                      acc_v.at[pl.ds(0, K)], ss).wait()

        def _seg_body(seg, seg_carry):
            seg_base = base_e + seg * SEG
            pltpu.sync_copy(idxi.at[pl.ds(seg_base, SEG)], idxi_b)
            pltpu.sync_copy(idxj.at[pl.ds(seg_base, SEG)], idxj_b)

            def _scan(g, cnt_vec):
                ii = idxi_b[pl.ds(g * GRP, GRP)]
                m = (ii >= lo) & (ii < lo + C)
                ones = jnp.where(m, jnp.int32(1), jnp.int32(0))
                x = ones
                for delta in (1, 2, 4, 8):
                    scanbuf[pl.ds(0, GRP)] = x
                    sh = plsc.load_gather(scanbuf,
                                          [jnp.maximum(iota16 - delta, 0)])
                    x = x + jnp.where(iota16 >= delta, sh, 0)
                scanbuf[pl.ds(0, GRP)] = x
                tot = plsc.load_gather(scanbuf, [splat15])
                pos = cnt_vec + x - ones
                tgt = jnp.where(m, pos, jnp.int32(SEG + GRP))
                plsc.store_scatter(sel_b, [tgt], g * GRP + iota16)
                return cnt_vec + tot

            cnt_vec = lax.fori_loop(0, SEG // GRP, _scan,
                                    jnp.zeros((GRP,), jnp.int32))
            cnt = cnt_vec[0]
            nbatch = (cnt + (K - 1)) // K

            def _issue(b, slot, gs):
                for g in range(K // GRP):
                    pos = b * K + g * GRP
                    valid = pos + iota16 < cnt
                    loc = sel_b[pl.ds(pos, GRP)]
                    locs = jnp.where(valid, loc, 0)
                    jj16 = plsc.load_gather(idxj_b, [locs])
                    jidx[slot, pl.ds(g * GRP, GRP)] = jnp.where(valid, jj16, 0)
                    eidx[slot, pl.ds(g * GRP, GRP)] = jnp.where(
                        valid, locs + seg_base, 0)
                pltpu.async_copy(w640.at[eidx.at[slot]], wbuf.at[slot], gs)
                pltpu.async_copy(cv.at[jidx.at[slot]], cvbuf.at[slot], gs)

            def _consume(b, slot, gs, ss):
                for g in range(K // GRP):
                    pos = b * K + g * GRP
                    valid = pos + iota16 < cnt
                    loc = sel_b[pl.ds(pos, GRP)]
                    locs = jnp.where(valid, loc, 0)
                    ii16 = plsc.load_gather(idxi_b, [locs])
                    rloc = jnp.where(valid, ii16 - lo, SCRAP)
                    ridx[slot, pl.ds(g * GRP, GRP)] = rloc
                    vr = 3 * rloc
                    vi0[slot, pl.ds(g * GRP, GRP)] = vr
                    vi1[slot, pl.ds(g * GRP, GRP)] = vr + 1
                    vi2[slot, pl.ds(g * GRP, GRP)] = vr + 2
                pltpu.make_async_copy(w640.at[pl.ds(0, K)],
                                      wbuf.at[slot], gs).wait()
                pltpu.make_async_copy(cv.at[pl.ds(0, K)],
                                      cvbuf.at[slot], gs).wait()

                def _edge(k, ecarry):
                    for h in range(H // GRP):
                        c1 = cvbuf[slot, k, pl.ds(h * GRP, GRP)]
                        w1v = wbuf[slot, k, pl.ds(h * GRP, GRP)]
                        osb[slot, k, pl.ds(h * GRP, GRP)] = c1 * w1v
                    d = [wbuf[slot, k, pl.ds(3 * H + cc * GRP, GRP)]
                         for cc in range(3)]
                    for h in range(H // GRP):
                        c2 = cvbuf[slot, k, pl.ds(H + h * GRP, GRP)]
                        c3 = cvbuf[slot, k, pl.ds(2 * H + h * GRP, GRP)]
                        wb = wbuf[slot, k, pl.ds(H + h * GRP, GRP)]
                        wc = wbuf[slot, k, pl.ds(2 * H + h * GRP, GRP)]
                        t = c3 * wc
                        cb = c2 * wb
                        for cc in range(3):
                            vj = cvbuf[slot, k,
                                       pl.ds(3 * H + cc * H + h * GRP, GRP)]
                            ovb[slot, cc, k, pl.ds(h * GRP, GRP)] = (
                                cb * d[cc] + t * vj)
                    return ecarry

                lax.fori_loop(0, K, _edge, 0)
                pltpu.async_copy(osb.at[slot], acc_s.at[ridx.at[slot]],
                                 ss, add=True)
                pltpu.async_copy(ovb.at[slot].at[0], acc_v.at[vi0.at[slot]],
                                 ss, add=True)
                pltpu.async_copy(ovb.at[slot].at[1], acc_v.at[vi1.at[slot]],
                                 ss, add=True)
                pltpu.async_copy(ovb.at[slot].at[2], acc_v.at[vi2.at[slot]],
                                 ss, add=True)

            @pl.when(nbatch > 0)
            def _prime():
                _issue(0, 0, gs0)

            def _pair(pidx, carry):
                p0, p1 = carry
                b0 = 2 * pidx
                b1 = b0 + 1

                @pl.when(b1 < nbatch)
                def _issue_b1():
                    _issue(b1, 1, gs1)

                @pl.when(p0 > 0)
                def _wait_s0():
                    _scat_wait(0, ss0)

                _consume(b0, 0, gs0, ss0)

                @pl.when(b1 + 1 < nbatch)
                def _issue_b2():
                    _issue(b1 + 1, 0, gs0)

                @pl.when(jnp.logical_and(b1 < nbatch, p1 > 0))
                def _wait_s1():
                    _scat_wait(1, ss1)

                @pl.when(b1 < nbatch)
                def _consume_b1():
                    _consume(b1, 1, gs1, ss1)

                return (jnp.int32(1),
                        jnp.where(b1 < nbatch, jnp.int32(1), p1))

            pend = lax.fori_loop(0, (nbatch + 1) // 2, _pair, seg_carry)
            return pend

        p0, p1 = lax.fori_loop(0, EPT // SEG, _seg_body,
                               (jnp.int32(0), jnp.int32(0)))

        @pl.when(p0 > 0)
        def _drain_s0():
            _scat_wait(0, ss0)

        @pl.when(p1 > 0)
        def _drain_s1():
            _scat_wait(1, ss1)

        plsc.subcore_barrier()

        full = jnp.logical_or(chunk < NCHUNK - 1, lo + r0 + RPT <= N)

        @pl.when(full)
        def _write_full():
            pltpu.sync_copy(acc_s.at[pl.ds(r0, RPT)],
                            su.at[pl.ds(lo + r0, RPT)])
            pltpu.sync_copy(acc_v.at[pl.ds(3 * r0, 3 * RPT)],
                            vu.at[pl.ds(3 * (lo + r0), 3 * RPT)])

        part = N % RPT
        if part:
            straddle = jnp.logical_and(chunk == NCHUNK - 1,
                                       jnp.logical_and(lo + r0 < N,
                                                       lo + r0 + RPT > N))

            @pl.when(straddle)
            def _write_part():
                pltpu.sync_copy(acc_s.at[pl.ds(r0, part)],
                                su.at[pl.ds(lo + r0, part)])
                pltpu.sync_copy(acc_v.at[pl.ds(3 * r0, 3 * part)],
                                vu.at[pl.ds(3 * (lo + r0), 3 * part)])

        plsc.subcore_barrier()
        return chunk_carry

    lax.fori_loop(0, NCHUNK // NC, _chunk_body, 0)


def _sc_scatter(w640, cv, idx_i, idx_j):
    mesh = plsc.VectorSubcoreMesh(core_axis_name="c", subcore_axis_name="s",
                                  num_cores=NC, num_subcores=NS)
    f = pl.kernel(
        _sc_body,
        compiler_params=pltpu.CompilerParams(needs_layout_passes=False),
        out_type=[
            jax.ShapeDtypeStruct((N, H), jnp.float32),
            jax.ShapeDtypeStruct((3 * N, H), jnp.float32),
        ],
        mesh=mesh,
        scratch_types=[
            pltpu.VMEM((SEG,), jnp.int32),
            pltpu.VMEM((SEG,), jnp.int32),
            pltpu.VMEM((SEG + 2 * GRP,), jnp.int32),
            pltpu.VMEM((GRP,), jnp.int32),
            pltpu.VMEM((2, K), jnp.int32),
            pltpu.VMEM((2, K), jnp.int32),
            pltpu.VMEM((2, K), jnp.int32),
            pltpu.VMEM((2, K), jnp.int32),
            pltpu.VMEM((2, K), jnp.int32),
            pltpu.VMEM((2, K), jnp.int32),
            pltpu.VMEM((2, K, W), jnp.float32),
            pltpu.VMEM((2, K, 6 * H), jnp.float32),
            pltpu.VMEM((2, K, H), jnp.float32),
            pltpu.VMEM((2, 3, K, H), jnp.float32),
            pltpu.VMEM_SHARED((C + 1, H), jnp.float32),
            pltpu.VMEM_SHARED((3 * (C + 1), H), jnp.float32),
            pltpu.SemaphoreType.DMA,
            pltpu.SemaphoreType.DMA,
            pltpu.SemaphoreType.DMA,
            pltpu.SemaphoreType.DMA,
        ],
    )
    return f(w640, cv, idx_i, idx_j)


def kernel(s, v, idx_i, idx_j, rbf, f_cut, dir_ij, inv_sqrt_neighbors,
           ln_w, ln_b, w1, b1, w2, b2, f1, fb1, f2, fb2):
    ctx = _ctx_tc(s, inv_sqrt_neighbors, ln_w, ln_b, w1, b1, w2, b2)
    w640 = _w640_tc(rbf, f_cut, dir_ij, f1, fb1, f2, fb2)
    cv = jnp.concatenate([ctx, v.reshape(N, 3 * H)], axis=1)
    su, vu = _sc_scatter(w640, cv, idx_i, idx_j)
    return s + su, v + vu.reshape(N, 3, H)


# speedup vs baseline: 1.0235x; 1.0235x over previous
import functools

import jax
import jax.numpy as jnp
from jax import lax
from jax.experimental import pallas as pl
from jax.experimental.pallas import tpu as pltpu
from jax.experimental.pallas import tpu_sc as plsc

N = 10000
E = 160000
H = 128
NRBF = 16

NC = 2
NS = 16
EPT = E // NS
NCHUNK = 8
C = 1280
RPT = C // NS
SCRAP = C
K = 16
GRP = 16
SEG = 2000
W = 512


def _silu(x):
    return x * (1.0 / (1.0 + jnp.exp(-x)))


def _ctx_body(inv_ref, lnw_ref, lnb_ref, w1t_ref, b1_ref, w2t_ref, b2_ref,
              s_ref, out_ref):
    x = s_ref[...]
    mu = jnp.mean(x, axis=1, keepdims=True)
    xc = x - mu
    var = jnp.mean(xc * xc, axis=1, keepdims=True)
    y = xc * lax.rsqrt(var + 1e-5) * lnw_ref[...] + lnb_ref[...]
    h1 = jnp.dot(y, w1t_ref[...], preferred_element_type=jnp.float32)
    h1 = h1 + b1_ref[...]
    h2 = jnp.dot(_silu(h1), w2t_ref[...], preferred_element_type=jnp.float32)
    out_ref[...] = (h2 + b2_ref[...]) * inv_ref[0, 0]


def _ctx_tc(s, inv, ln_w, ln_b, w1, b1, w2, b2):
    BA = 400
    grid = N // BA
    return pl.pallas_call(
        _ctx_body,
        grid=(grid,),
        in_specs=[
            pl.BlockSpec(memory_space=pltpu.SMEM),
            pl.BlockSpec((1, H), lambda i: (0, 0)),
            pl.BlockSpec((1, H), lambda i: (0, 0)),
            pl.BlockSpec((H, H), lambda i: (0, 0)),
            pl.BlockSpec((1, H), lambda i: (0, 0)),
            pl.BlockSpec((H, 3 * H), lambda i: (0, 0)),
            pl.BlockSpec((1, 3 * H), lambda i: (0, 0)),
            pl.BlockSpec((BA, H), lambda i: (i, 0)),
        ],
        out_specs=pl.BlockSpec((BA, 3 * H), lambda i: (i, 0)),
        out_shape=jax.ShapeDtypeStruct((N, 3 * H), jnp.float32),
    )(inv.reshape(1, 1), ln_w.reshape(1, H), ln_b.reshape(1, H),
      w1.T, b1.reshape(1, H), w2.T, b2.reshape(1, 3 * H), s)


def _w640_body(f1t_ref, fb1_ref, f2t_ref, fb2_ref, rbf_ref, fc_ref, dir_ref,
               out_ref):
    h1 = jnp.dot(rbf_ref[...], f1t_ref[...],
                 preferred_element_type=jnp.float32) + fb1_ref[...]
    g = jnp.dot(_silu(h1), f2t_ref[...],
                preferred_element_type=jnp.float32) + fb2_ref[...]
    fc = fc_ref[...]
    ga = g[:, :H] * fc
    gb = g[:, H:2 * H] * fc
    gc = g[:, 2 * H:] * fc
    be = ga.shape[0]
    d0 = jnp.broadcast_to(dir_ref[:, 0:1], (be, GRP))
    d1 = jnp.broadcast_to(dir_ref[:, 1:2], (be, GRP))
    d2 = jnp.broadcast_to(dir_ref[:, 2:3], (be, GRP))
    pad = jnp.zeros((be, W - 3 * H - 3 * GRP), jnp.float32)
    out_ref[...] = jnp.concatenate([ga, gb, gc, d0, d1, d2, pad], axis=1)


def _w640_tc(rbf, f_cut, dir_ij, f1, fb1, f2, fb2):
    BE = 1000
    grid = E // BE
    return pl.pallas_call(
        _w640_body,
        grid=(grid,),
        in_specs=[
            pl.BlockSpec((NRBF, H), lambda i: (0, 0)),
            pl.BlockSpec((1, H), lambda i: (0, 0)),
            pl.BlockSpec((H, 3 * H), lambda i: (0, 0)),
            pl.BlockSpec((1, 3 * H), lambda i: (0, 0)),
            pl.BlockSpec((BE, NRBF), lambda i: (i, 0)),
            pl.BlockSpec((BE, 1), lambda i: (i, 0)),
            pl.BlockSpec((BE, 3), lambda i: (i, 0)),
        ],
        out_specs=pl.BlockSpec((BE, W), lambda i: (i, 0)),
        out_shape=jax.ShapeDtypeStruct((E, W), jnp.float32),
    )(f1.T, fb1.reshape(1, H), f2.T, fb2.reshape(1, 3 * H),
      rbf, f_cut.reshape(E, 1), dir_ij)


def _sc_body(w640, cv, idxi, idxj, out,
             idxi_b, idxj_b, sel_b, scanbuf, jidx, eidx, sidx,
             wbuf, cvbuf, mb, acc, gs0, gs1, ss0, ss1):
    c = lax.axis_index("c")
    s = lax.axis_index("s")
    base_e = s * EPT
    z16 = jnp.zeros((GRP,), jnp.float32)
    iota16 = lax.iota(jnp.int32, GRP)

    def _chunk_body(pc, chunk_carry):
        chunk = c * (NCHUNK // NC) + pc
        lo = chunk * C
        r0 = s * RPT

        def _zero_row(i, carry):
            for h in range(H // GRP):
                mb[0, i, pl.ds(h * GRP, GRP)] = z16
            return carry

        lax.fori_loop(0, 4 * K, _zero_row, 0)

        def _copies(total, step):
            o, off = [], 0
            while off < total:
                n = min(step, total - off)
                o.append((off, n))
                off += n
            return o

        for off, n in _copies(4 * RPT, 4 * K):
            pltpu.sync_copy(mb.at[0].at[pl.ds(0, n)],
                            acc.at[pl.ds(4 * r0 + off, n)])

        @pl.when(s == 0)
        def _zero_scrap():
            pltpu.sync_copy(mb.at[0].at[pl.ds(0, 4)],
                            acc.at[pl.ds(4 * SCRAP, 4)])

        plsc.subcore_barrier()

        splat15 = jnp.full((GRP,), 15, jnp.int32)

        def _scat_wait(slot, ss):
            pltpu.make_async_copy(mb.at[slot],
                                  acc.at[pl.ds(0, 4 * K)], ss).wait()

        def _seg_body(seg, seg_carry):
            seg_base = base_e + seg * SEG
            pltpu.sync_copy(idxi.at[pl.ds(seg_base, SEG)], idxi_b)
            pltpu.sync_copy(idxj.at[pl.ds(seg_base, SEG)], idxj_b)

            def _scan(g, cnt_vec):
                ii = idxi_b[pl.ds(g * GRP, GRP)]
                m = (ii >= lo) & (ii < lo + C)
                ones = jnp.where(m, jnp.int32(1), jnp.int32(0))
                x = ones
                for delta in (1, 2, 4, 8):
                    scanbuf[pl.ds(0, GRP)] = x
                    sh = plsc.load_gather(scanbuf,
                                          [jnp.maximum(iota16 - delta, 0)])
                    x = x + jnp.where(iota16 >= delta, sh, 0)
                scanbuf[pl.ds(0, GRP)] = x
                tot = plsc.load_gather(scanbuf, [splat15])
                pos = cnt_vec + x - ones
                tgt = jnp.where(m, pos, jnp.int32(SEG + GRP))
                plsc.store_scatter(sel_b, [tgt], g * GRP + iota16)
                return cnt_vec + tot

            cnt_vec = lax.fori_loop(0, SEG // GRP, _scan,
                                    jnp.zeros((GRP,), jnp.int32))
            cnt = cnt_vec[0]
            nbatch = (cnt + (K - 1)) // K

            def _issue(b, slot, gs):
                for g in range(K // GRP):
                    pos = b * K + g * GRP
                    valid = pos + iota16 < cnt
                    loc = sel_b[pl.ds(pos, GRP)]
                    locs = jnp.where(valid, loc, 0)
                    jj16 = plsc.load_gather(idxj_b, [locs])
                    jidx[slot, pl.ds(g * GRP, GRP)] = jnp.where(valid, jj16, 0)
                    eidx[slot, pl.ds(g * GRP, GRP)] = jnp.where(
                        valid, locs + seg_base, 0)
                pltpu.async_copy(w640.at[eidx.at[slot]], wbuf.at[slot], gs)
                pltpu.async_copy(cv.at[jidx.at[slot]], cvbuf.at[slot], gs)

            def _consume(b, slot, gs, ss):
                for g in range(K // GRP):
                    pos = b * K + g * GRP
                    valid = pos + iota16 < cnt
                    loc = sel_b[pl.ds(pos, GRP)]
                    locs = jnp.where(valid, loc, 0)
                    ii16 = plsc.load_gather(idxi_b, [locs])
                    rloc = jnp.where(valid, ii16 - lo, SCRAP)
                    r4 = 4 * rloc
                    sidx[slot, pl.ds(0 * K + g * GRP, GRP)] = r4
                    sidx[slot, pl.ds(1 * K + g * GRP, GRP)] = r4 + 1
                    sidx[slot, pl.ds(2 * K + g * GRP, GRP)] = r4 + 2
                    sidx[slot, pl.ds(3 * K + g * GRP, GRP)] = r4 + 3
                pltpu.make_async_copy(w640.at[pl.ds(0, K)],
                                      wbuf.at[slot], gs).wait()
                pltpu.make_async_copy(cv.at[pl.ds(0, K)],
                                      cvbuf.at[slot], gs).wait()

                def _edge(k, ecarry):
                    for h in range(H // GRP):
                        c1 = cvbuf[slot, k, pl.ds(h * GRP, GRP)]
                        w1v = wbuf[slot, k, pl.ds(h * GRP, GRP)]
                        mb[slot, k, pl.ds(h * GRP, GRP)] = c1 * w1v
                    d = [wbuf[slot, k, pl.ds(3 * H + cc * GRP, GRP)]
                         for cc in range(3)]
                    for h in range(H // GRP):
                        c2 = cvbuf[slot, k, pl.ds(H + h * GRP, GRP)]
                        c3 = cvbuf[slot, k, pl.ds(2 * H + h * GRP, GRP)]
                        wb = wbuf[slot, k, pl.ds(H + h * GRP, GRP)]
                        wc = wbuf[slot, k, pl.ds(2 * H + h * GRP, GRP)]
                        t = c3 * wc
                        cb = c2 * wb
                        for cc in range(3):
                            vj = cvbuf[slot, k,
                                       pl.ds(3 * H + cc * H + h * GRP, GRP)]
                            mb[slot, (1 + cc) * K + k, pl.ds(h * GRP, GRP)] = (
                                cb * d[cc] + t * vj)
                    return ecarry

                lax.fori_loop(0, K, _edge, 0)
                pltpu.async_copy(mb.at[slot], acc.at[sidx.at[slot]],
                                 ss, add=True)

            @pl.when(nbatch > 0)
            def _prime():
                _issue(0, 0, gs0)

            def _pair(pidx, carry):
                p0, p1 = carry
                b0 = 2 * pidx
                b1 = b0 + 1

                @pl.when(b1 < nbatch)
                def _issue_b1():
                    _issue(b1, 1, gs1)

                @pl.when(p0 > 0)
                def _wait_s0():
                    _scat_wait(0, ss0)

                _consume(b0, 0, gs0, ss0)

                @pl.when(b1 + 1 < nbatch)
                def _issue_b2():
                    _issue(b1 + 1, 0, gs0)

                @pl.when(jnp.logical_and(b1 < nbatch, p1 > 0))
                def _wait_s1():
                    _scat_wait(1, ss1)

                @pl.when(b1 < nbatch)
                def _consume_b1():
                    _consume(b1, 1, gs1, ss1)

                return (jnp.int32(1),
                        jnp.where(b1 < nbatch, jnp.int32(1), p1))

            pend = lax.fori_loop(0, (nbatch + 1) // 2, _pair, seg_carry)
            return pend

        p0, p1 = lax.fori_loop(0, EPT // SEG, _seg_body,
                               (jnp.int32(0), jnp.int32(0)))

        @pl.when(p0 > 0)
        def _drain_s0():
            _scat_wait(0, ss0)

        @pl.when(p1 > 0)
        def _drain_s1():
            _scat_wait(1, ss1)

        plsc.subcore_barrier()

        full = jnp.logical_or(chunk < NCHUNK - 1, lo + r0 + RPT <= N)

        @pl.when(full)
        def _write_full():
            pltpu.sync_copy(acc.at[pl.ds(4 * r0, 4 * RPT)],
                            out.at[pl.ds(4 * (lo + r0), 4 * RPT)])

        part = N % RPT
        if part:
            straddle = jnp.logical_and(chunk == NCHUNK - 1,
                                       jnp.logical_and(lo + r0 < N,
                                                       lo + r0 + RPT > N))

            @pl.when(straddle)
            def _write_part():
                pltpu.sync_copy(acc.at[pl.ds(4 * r0, 4 * part)],
                                out.at[pl.ds(4 * (lo + r0), 4 * part)])

        plsc.subcore_barrier()
        return chunk_carry

    lax.fori_loop(0, NCHUNK // NC, _chunk_body, 0)


def _sc_scatter(w640, cv, idx_i, idx_j):
    mesh = plsc.VectorSubcoreMesh(core_axis_name="c", subcore_axis_name="s",
                                  num_cores=NC, num_subcores=NS)
    f = pl.kernel(
        _sc_body,
        compiler_params=pltpu.CompilerParams(needs_layout_passes=False),
        out_type=[
            jax.ShapeDtypeStruct((4 * N, H), jnp.float32),
        ],
        mesh=mesh,
        scratch_types=[
            pltpu.VMEM((SEG,), jnp.int32),
            pltpu.VMEM((SEG,), jnp.int32),
            pltpu.VMEM((SEG + 2 * GRP,), jnp.int32),
            pltpu.VMEM((GRP,), jnp.int32),
            pltpu.VMEM((2, K), jnp.int32),
            pltpu.VMEM((2, K), jnp.int32),
            pltpu.VMEM((2, 4 * K), jnp.int32),
            pltpu.VMEM((2, K, W), jnp.float32),
            pltpu.VMEM((2, K, 6 * H), jnp.float32),
            pltpu.VMEM((2, 4 * K, H), jnp.float32),
            pltpu.VMEM_SHARED((4 * (C + 1), H), jnp.float32),
            pltpu.SemaphoreType.DMA,
            pltpu.SemaphoreType.DMA,
            pltpu.SemaphoreType.DMA,
            pltpu.SemaphoreType.DMA,
        ],
    )
    return f(w640, cv, idx_i, idx_j)


def kernel(s, v, idx_i, idx_j, rbf, f_cut, dir_ij, inv_sqrt_neighbors,
           ln_w, ln_b, w1, b1, w2, b2, f1, fb1, f2, fb2):
    ctx = _ctx_tc(s, inv_sqrt_neighbors, ln_w, ln_b, w1, b1, w2, b2)
    w640 = _w640_tc(rbf, f_cut, dir_ij, f1, fb1, f2, fb2)
    cv = jnp.concatenate([ctx, v.reshape(N, 3 * H)], axis=1)
    (out,) = _sc_scatter(w640, cv, idx_i, idx_j)
    o4 = out.reshape(N, 4, H)
    return s + o4[:, 0], v + o4[:, 1:4]
